# Initial kernel scaffold; baseline (speedup 1.0000x reference)
import jax
import jax.numpy as jnp
from jax.experimental import pallas as pl


def kernel(pos_embed, num_queries):
    raise NotImplementedError("write your pallas kernel here")


# speedup vs baseline: 1.5165x; 1.5165x over previous
import functools

import jax
import jax.numpy as jnp
from jax import lax
from jax.experimental import pallas as pl
from jax.experimental.pallas import tpu as pltpu
from jax.experimental.pallas import tpu_sc as plsc

NUM_WORKERS = 32
LANES = 16


def _lookup_call(n, d, chunk_rows, nbuf):
    rows_per_w = n // NUM_WORKERS
    num_chunks = rows_per_w // chunk_rows
    mesh = plsc.VectorSubcoreMesh(core_axis_name="c", subcore_axis_name="s")

    @functools.partial(
        pl.kernel,
        out_type=jax.ShapeDtypeStruct((n, d), jnp.float32),
        mesh=mesh,
        scratch_types=[
            pltpu.VMEM((LANES,), jnp.int32),
            pltpu.VMEM((nbuf, chunk_rows), jnp.int32),
            pltpu.VMEM((nbuf, chunk_rows, d), jnp.float32),
            [pltpu.SemaphoreType.DMA] * nbuf,
            [pltpu.SemaphoreType.DMA] * nbuf,
        ],
    )
    def k(table_hbm, maxidx_hbm, out_hbm, maxidx_v, idx_v, rows_v, gsems,
          wsems):
        wid = lax.axis_index("s") * 2 + lax.axis_index("c")
        base = wid * rows_per_w
        pltpu.sync_copy(maxidx_hbm, maxidx_v)
        maxidx = maxidx_v[...]

        def fill_idx(b, chunk_start):
            for j in range(chunk_rows // LANES):
                ramp = lax.iota(jnp.int32, LANES) + (chunk_start + j * LANES)
                idx_v[b, pl.ds(j * LANES, LANES)] = jnp.minimum(ramp, maxidx)

        def start_gather(b, c):
            fill_idx(b, base + c * chunk_rows)
            return pltpu.async_copy(table_hbm.at[idx_v.at[b]], rows_v.at[b],
                                    gsems[b])

        def start_write(b, c):
            return pltpu.async_copy(
                rows_v.at[b], out_hbm.at[pl.ds(base + c * chunk_rows,
                                               chunk_rows)], wsems[b])

        gathers = [start_gather(b, b) for b in range(min(nbuf, num_chunks))]
        writes = [None] * nbuf
        for c in range(num_chunks):
            b = c % nbuf
            gathers[b].wait()
            if writes[b] is not None:
                writes[b].wait()
            writes[b] = start_write(b, c)
            nxt = c + nbuf
            if nxt < num_chunks:
                writes[b].wait()
                writes[b] = None
                gathers[b] = start_gather(b, nxt)
        for w in writes:
            if w is not None:
                w.wait()

    return k


def kernel(pos_embed, num_queries):
    n, d = pos_embed.shape
    maxidx = jnp.full((LANES,), num_queries, jnp.int32) - 1
    out = _lookup_call(n, d, chunk_rows=32, nbuf=2)(pos_embed, maxidx)
    return out[None]
